# baseline (speedup 1.0000x reference)
import jax
import jax.numpy as jnp
from jax import lax
from jax.experimental import pallas as pl
from jax.experimental.pallas import tpu as pltpu
from jax.experimental.pallas import tpu_sc as plsc

B, L = 1024, 200
N = B * L
WORD_D = 64
AUX_D = 32
OUT_D = WORD_D + 4 * AUX_D

NUM_CORES = 2
NUM_SUBCORES = 16
NW = NUM_CORES * NUM_SUBCORES
PER_W = N // NW
CHUNK = 128
NCHUNK = PER_W // CHUNK

_COL_OFF = (0, WORD_D, WORD_D + AUX_D, WORD_D + 2 * AUX_D, WORD_D + 3 * AUX_D)
_DIMS = (WORD_D, AUX_D, AUX_D, AUX_D, AUX_D)


def _emb_kernel(word_hbm, pos_hbm, ner_hbm, deprel_hbm, position_hbm,
                widx_hbm, pidx_hbm, nidx_hbm, didx_hbm, xidx_hbm,
                out_hbm,
                widx_v, pidx_v, nidx_v, didx_v, xidx_v,
                rows0_w, rows0_p, rows0_n, rows0_d, rows0_x,
                rows1_w, rows1_p, rows1_n, rows1_d, rows1_x,
                gsem0, gsem1, wsem0, wsem1):
    wid = lax.axis_index("s") * NUM_CORES + lax.axis_index("c")
    crow = wid * NCHUNK

    pltpu.sync_copy(widx_hbm.at[pl.ds(crow, NCHUNK)], widx_v)
    pltpu.sync_copy(pidx_hbm.at[pl.ds(crow, NCHUNK)], pidx_v)
    pltpu.sync_copy(nidx_hbm.at[pl.ds(crow, NCHUNK)], nidx_v)
    pltpu.sync_copy(didx_hbm.at[pl.ds(crow, NCHUNK)], didx_v)
    pltpu.sync_copy(xidx_hbm.at[pl.ds(crow, NCHUNK)], xidx_v)

    tables = (word_hbm, pos_hbm, ner_hbm, deprel_hbm, position_hbm)
    idxs = (widx_v, pidx_v, nidx_v, didx_v, xidx_v)
    rows = ((rows0_w, rows0_p, rows0_n, rows0_d, rows0_x),
            (rows1_w, rows1_p, rows1_n, rows1_d, rows1_x))
    gsems = (gsem0, gsem1)
    wsems = (wsem0, wsem1)

    gh = [None, None]
    wh = [None, None]
    for i in range(NCHUNK + 1):
        if i < NCHUNK:
            s = i % 2
            if wh[s] is not None:
                for h in wh[s]:
                    h.wait()
                wh[s] = None
            gh[s] = [
                pltpu.async_copy(tab.at[idx.at[i]], buf, gsems[s])
                for tab, idx, buf in zip(tables, idxs, rows[s])
            ]
        if i >= 1:
            j = i - 1
            s2 = j % 2
            for h in gh[s2]:
                h.wait()
            row0 = (crow + j) * CHUNK
            wh[s2] = [
                pltpu.async_copy(
                    buf, out_hbm.at[pl.ds(row0, CHUNK), pl.ds(off, dim)],
                    wsems[s2])
                for buf, off, dim in zip(rows[s2], _COL_OFF, _DIMS)
            ]
    for s in (0, 1):
        if wh[s] is not None:
            for h in wh[s]:
                h.wait()


@jax.jit
def kernel(word_table, pos_table, ner_table, deprel_table, position_table,
           word_rep, pos_rep, ner_rep, deprel_rep, position_rep):
    mesh = plsc.VectorSubcoreMesh(core_axis_name="c", subcore_axis_name="s")
    run = pl.kernel(
        _emb_kernel,
        out_type=jax.ShapeDtypeStruct((N, OUT_D), jnp.float32),
        mesh=mesh,
        compiler_params=pltpu.CompilerParams(use_tc_tiling_on_sc=False),
        scratch_types=(
            [pltpu.VMEM((NCHUNK, CHUNK), jnp.int32) for _ in range(5)]
            + [pltpu.VMEM((CHUNK, d), jnp.float32) for d in _DIMS]
            + [pltpu.VMEM((CHUNK, d), jnp.float32) for d in _DIMS]
            + [pltpu.SemaphoreType.DMA] * 4
        ),
    )
    out = run(
        word_table, pos_table, ner_table, deprel_table, position_table,
        word_rep.reshape(N // CHUNK, CHUNK).astype(jnp.int32),
        pos_rep.reshape(N // CHUNK, CHUNK).astype(jnp.int32),
        ner_rep.reshape(N // CHUNK, CHUNK).astype(jnp.int32),
        deprel_rep.reshape(N // CHUNK, CHUNK).astype(jnp.int32),
        position_rep.reshape(N // CHUNK, CHUNK).astype(jnp.int32),
    )
    return out.reshape(B, L, OUT_D)
